# baseline (speedup 1.0000x reference)
import jax
import jax.numpy as jnp
from jax.experimental import pallas as pl

_DIM = 32
_NUM_Q = 8
_K = 1024
_COMMIT = 0.25
_BLOCK_B = 1024
_N_CHUNK = 2


def _split_bf16_exact(w):
    def trunc(v):
        bits = jax.lax.bitcast_convert_type(v, jnp.uint32)
        return jax.lax.bitcast_convert_type(bits & jnp.uint32(0xFFFF0000),
                                            jnp.float32)
    a = trunc(w)
    rem1 = w - a
    b = trunc(rem1)
    c = rem1 - b
    return a, b, c


def _rvq_body(x_ref, wsplit_ref, wt2_ref, w2_ref, quant_ref, idx_ref, loss_ref):
    cb = _BLOCK_B // _N_CHUNK
    rs = [x_ref[c * cb:(c + 1) * cb, :] for c in range(_N_CHUNK)]
    qs = [[] for _ in range(_N_CHUNK)]
    idxs = [[] for _ in range(_N_CHUNK)]
    part = jnp.zeros((), jnp.float32)
    for s in range(_NUM_Q):
        wt2 = wt2_ref[s]
        wsp = wsplit_ref[s]
        w2 = w2_ref[s]
        m2 = [jnp.dot(rs[c].astype(jnp.bfloat16), wt2,
                      preferred_element_type=jnp.float32)
              for c in range(_N_CHUNK)]
        for c in range(_N_CHUNK):
            r = rs[c]
            r2 = jnp.sum(r * r, axis=1, keepdims=True)
            score = (r2 + w2) - m2[c]
            idx = jnp.argmin(score, axis=1)
            onehot = (jax.lax.broadcasted_iota(jnp.int32, score.shape, 1)
                      == idx[:, None]).astype(jnp.bfloat16)
            qcat = jnp.dot(onehot, wsp,
                           preferred_element_type=jnp.float32)
            q = (qcat[:, :_DIM] + qcat[:, _DIM:2 * _DIM]) + qcat[:, 2 * _DIM:]
            qs[c].append(r + (q - r))
            idxs[c].append(idx)
            rs[c] = r - q
            cc = _COMMIT if s < _NUM_Q - 1 else 1.0 + _COMMIT
            part = part + cc * jnp.sum(rs[c] * rs[c])
    for c in range(_N_CHUNK):
        sl = slice(c * cb, (c + 1) * cb)
        quant_ref[sl, :, :] = jnp.stack(qs[c], axis=1)
        idx_ref[sl, :] = jnp.stack(idxs[c], axis=1).astype(jnp.int32)

    @pl.when(pl.program_id(0) == 0)
    def _():
        loss_ref[...] = jnp.zeros((1, 1), jnp.float32)

    loss_ref[...] = loss_ref[...] + part.reshape(1, 1)


def kernel(x, W):
    B, D = x.shape
    Q, K, _ = W.shape
    Wt2 = (2.0 * jnp.swapaxes(W, 1, 2)).astype(jnp.bfloat16)
    W2 = jnp.sum(W * W, axis=2)[:, None, :]
    wa, wb, wc = _split_bf16_exact(W)
    Wsplit = jnp.concatenate([wa, wb, wc], axis=2).astype(jnp.bfloat16)
    n_blocks = B // _BLOCK_B
    quant, idx, loss = pl.pallas_call(
        _rvq_body,
        grid=(n_blocks,),
        in_specs=[
            pl.BlockSpec((_BLOCK_B, D), lambda i: (i, 0)),
            pl.BlockSpec((Q, K, 3 * D), lambda i: (0, 0, 0)),
            pl.BlockSpec((Q, D, K), lambda i: (0, 0, 0)),
            pl.BlockSpec((Q, 1, K), lambda i: (0, 0, 0)),
        ],
        out_specs=[
            pl.BlockSpec((_BLOCK_B, Q, D), lambda i: (i, 0, 0)),
            pl.BlockSpec((_BLOCK_B, Q), lambda i: (i, 0)),
            pl.BlockSpec((1, 1), lambda i: (0, 0)),
        ],
        out_shape=[
            jax.ShapeDtypeStruct((B, Q, D), jnp.float32),
            jax.ShapeDtypeStruct((B, Q), jnp.int32),
            jax.ShapeDtypeStruct((1, 1), jnp.float32),
        ],
    )(x, Wsplit, Wt2, W2)
    return quant, idx, loss[0, 0] / (B * D)


# speedup vs baseline: 1.3364x; 1.3364x over previous
import functools

import jax
import jax.numpy as jnp
from jax import lax
from jax.experimental import pallas as pl
from jax.experimental.pallas import tpu as pltpu
from jax.experimental.pallas import tpu_sc as plsc

_DIM = 32
_NUM_Q = 8
_K = 1024
_COMMIT = 0.25
_BLOCK_B = 1024
_B = 32768


def _stage_body(r_in_ref, q_in_ref, wt2_ref, w2_ref,
                r_ref, idx_ref, qste_ref, loss_ref):
    r_in = r_in_ref[...]
    q_in = q_in_ref[:, :_DIM]
    r = r_in - q_in
    qste_ref[...] = r_in + (q_in - r_in)
    m2 = jnp.dot(r.astype(jnp.bfloat16), wt2_ref[...],
                 preferred_element_type=jnp.float32)
    r2 = jnp.sum(r * r, axis=1, keepdims=True)
    score = (r2 + w2_ref[...]) - m2
    idx_ref[...] = jnp.argmin(score, axis=1)[:, None]
    r_ref[...] = r

    @pl.when(pl.program_id(0) == 0)
    def _():
        loss_ref[...] = jnp.zeros((1, 1), jnp.float32)

    loss_ref[...] = loss_ref[...] + jnp.sum(r * r).reshape(1, 1)


def _finish_body(r_in_ref, q_in_ref, qste_ref, loss_ref):
    r_in = r_in_ref[...]
    q_in = q_in_ref[:, :_DIM]
    r = r_in - q_in
    qste_ref[...] = r_in + (q_in - r_in)

    @pl.when(pl.program_id(0) == 0)
    def _():
        loss_ref[...] = jnp.zeros((1, 1), jnp.float32)

    loss_ref[...] = loss_ref[...] + jnp.sum(r * r).reshape(1, 1)


def _tc_stage(r_in, q_in, wt2, w2):
    n_blocks = _B // _BLOCK_B
    return pl.pallas_call(
        _stage_body,
        grid=(n_blocks,),
        in_specs=[
            pl.BlockSpec((_BLOCK_B, _DIM), lambda i: (i, 0)),
            pl.BlockSpec((_BLOCK_B, 128), lambda i: (i, 0)),
            pl.BlockSpec((_DIM, _K), lambda i: (0, 0)),
            pl.BlockSpec((1, _K), lambda i: (0, 0)),
        ],
        out_specs=[
            pl.BlockSpec((_BLOCK_B, _DIM), lambda i: (i, 0)),
            pl.BlockSpec((_BLOCK_B, 1), lambda i: (i, 0)),
            pl.BlockSpec((_BLOCK_B, _DIM), lambda i: (i, 0)),
            pl.BlockSpec((1, 1), lambda i: (0, 0)),
        ],
        out_shape=[
            jax.ShapeDtypeStruct((_B, _DIM), jnp.float32),
            jax.ShapeDtypeStruct((_B, 1), jnp.int32),
            jax.ShapeDtypeStruct((_B, _DIM), jnp.float32),
            jax.ShapeDtypeStruct((1, 1), jnp.float32),
        ],
    )(r_in, q_in, wt2, w2)


def _tc_finish(r_in, q_in):
    n_blocks = _B // _BLOCK_B
    return pl.pallas_call(
        _finish_body,
        grid=(n_blocks,),
        in_specs=[
            pl.BlockSpec((_BLOCK_B, _DIM), lambda i: (i, 0)),
            pl.BlockSpec((_BLOCK_B, 128), lambda i: (i, 0)),
        ],
        out_specs=[
            pl.BlockSpec((_BLOCK_B, _DIM), lambda i: (i, 0)),
            pl.BlockSpec((1, 1), lambda i: (0, 0)),
        ],
        out_shape=[
            jax.ShapeDtypeStruct((_B, _DIM), jnp.float32),
            jax.ShapeDtypeStruct((1, 1), jnp.float32),
        ],
    )(r_in, q_in)


def _make_sc_gather():
    info = plsc.get_sparse_core_info()
    nc, ns = info.num_cores, info.num_subcores
    nw = nc * ns
    bpw = _B // nw
    n_sub = bpw // 128
    half = n_sub // 2
    mesh = plsc.VectorSubcoreMesh(core_axis_name="c", subcore_axis_name="s")

    @functools.partial(
        pl.kernel, mesh=mesh,
        out_type=jax.ShapeDtypeStruct((_B, 128), jnp.float32),
        scratch_types=[
            pltpu.VMEM((n_sub, 128), jnp.int32),
            pltpu.VMEM((half * 128, 128), jnp.float32),
            pltpu.SemaphoreType.DMA,
        ],
    )
    def gather(table_hbm, idx_hbm, out_hbm, idx_v, rows_v, sem):
        wid = lax.axis_index("s") * nc + lax.axis_index("c")
        base = wid * bpw
        pltpu.sync_copy(idx_hbm.at[wid], idx_v)
        for h in range(2):
            cps = []
            for j in range(half):
                cps.append(pltpu.async_copy(
                    table_hbm.at[idx_v.at[h * half + j]],
                    rows_v.at[pl.ds(j * 128, 128)], sem))
            for cp in cps:
                cp.wait()
            pltpu.sync_copy(
                rows_v, out_hbm.at[pl.ds(base + h * half * 128, half * 128)])

    return gather, nw, bpw, n_sub


def kernel(x, W):
    B, D = x.shape
    Q, K, _ = W.shape
    Wt2 = (2.0 * jnp.swapaxes(W, 1, 2)).astype(jnp.bfloat16)
    W2 = jnp.sum(W * W, axis=2)[:, None, :]

    sc_gather, nw, bpw, n_sub = _make_sc_gather()

    Wpad = jnp.pad(W, ((0, 0), (0, 0), (0, 128 - D)))
    qstes = [None] * Q
    losses = []
    q = jnp.zeros((B, 128), jnp.float32)
    r = x
    idxs = []
    for s in range(Q):
        r, idx2d, qste, lpart = _tc_stage(r, q, Wt2[s], W2[s])
        if s > 0:
            qstes[s - 1] = qste
            losses.append(lpart[0, 0])
        idxs.append(idx2d[:, 0])
        idx_sc = idx2d.reshape(nw, n_sub, 128)
        q = sc_gather(Wpad[s], idx_sc)
    qste_last, lpart = _tc_finish(r, q)
    qstes[Q - 1] = qste_last
    l_last = lpart[0, 0]
    commit_sum = l_last
    for lp in losses:
        commit_sum = commit_sum + lp
    loss = (l_last + _COMMIT * commit_sum) / (B * D)
    quantized = jnp.stack(qstes, axis=1)
    indices = jnp.stack(idxs, axis=1)
    return quantized, indices, loss
